# baseline (speedup 1.0000x reference)
import functools

import jax
import jax.numpy as jnp
from jax import lax
from jax.experimental import pallas as pl
from jax.experimental.pallas import tpu as pltpu
from jax.experimental.pallas import tpu_sc as plsc

_THRESHOLD = jnp.float32(0.3)

_B, _D, _C = 16, 20000, 6
_TOTAL = _B * _D * _C
_NC, _NS = 2, 16
_NW = _NC * _NS
_PER_W = _TOTAL // _NW
_L = 16
_GROUP = 48


def _make_kernel():
    mesh = plsc.VectorSubcoreMesh(core_axis_name="c", subcore_axis_name="s")

    @functools.partial(
        pl.kernel,
        mesh=mesh,
        out_type=jax.ShapeDtypeStruct((_TOTAL,), jnp.float32),
        scratch_types=[pltpu.VMEM((_PER_W,), jnp.float32)],
        compiler_params=pltpu.CompilerParams(needs_layout_passes=False),
    )
    def _filter(x_hbm, out_hbm, buf):
        wid = lax.axis_index("s") * _NC + lax.axis_index("c")
        off = wid * _PER_W
        pltpu.sync_copy(x_hbm.at[pl.ds(off, _PER_W)], buf)

        iota = lax.iota(jnp.int32, _L)
        pats = [iota - (iota + (16 * k) % 6) % 6 + 1 for k in range(3)]

        def body(g, _):
            gb = g * _GROUP
            for k in range(3):
                vb = gb + k * _L
                idx = pats[k] + vb
                conf = plsc.load_gather(buf, [idx])
                x = buf[pl.ds(vb, _L)]
                buf[pl.ds(vb, _L)] = jnp.where(conf > _THRESHOLD, x,
                                               jnp.float32(0.0))
            return 0

        lax.fori_loop(0, _PER_W // _GROUP, body, 0)
        pltpu.sync_copy(buf, out_hbm.at[pl.ds(off, _PER_W)])

    return _filter


_FILTER = _make_kernel()


@jax.jit
def kernel(boxes):
    flat = boxes.reshape(_TOTAL)
    out = _FILTER(flat)
    return out.reshape(_B, _D, _C)


# speedup vs baseline: 1.0366x; 1.0366x over previous
import functools

import numpy as np

import jax
import jax.numpy as jnp
from jax import lax
from jax.experimental import pallas as pl
from jax.experimental.pallas import tpu as pltpu
from jax.experimental.pallas import tpu_sc as plsc

_THRESHOLD = jnp.float32(0.3)

_B, _D, _C = 16, 20000, 6
_TOTAL = _B * _D * _C
_NC, _NS = 2, 16
_NW = _NC * _NS
_PER_W = _TOTAL // _NW
_L = 16
_GROUP = 48


def _make_kernel():
    mesh = plsc.VectorSubcoreMesh(core_axis_name="c", subcore_axis_name="s")

    @functools.partial(
        pl.kernel,
        mesh=mesh,
        out_type=jax.ShapeDtypeStruct((_TOTAL,), jnp.float32),
        scratch_types=[
            pltpu.VMEM((_PER_W,), jnp.float32),
            pltpu.VMEM((_PER_W,), jnp.float32),
        ],
        compiler_params=pltpu.CompilerParams(needs_layout_passes=False),
    )
    def _filter(x_hbm, out_hbm, buf, obuf):
        wid = lax.axis_index("s") * _NC + lax.axis_index("c")
        off = wid * _PER_W
        pltpu.sync_copy(x_hbm.at[pl.ds(off, _PER_W)], buf)

        lane = lax.iota(jnp.int32, _L)

        def cidx(k, src):
            c = 6 * ((lane + 16 * k) // 6) + 1 - _L * src
            return jnp.clip(c, 0, _L - 1)

        i00 = cidx(0, 0)
        i10, i11 = cidx(1, 0), cidx(1, 1)
        i21, i22 = cidx(2, 1), cidx(2, 2)
        sel1 = lane < 2
        sel2 = lane < 4

        def take(x, i):
            return jnp.take_along_axis(
                x, i, axis=0, mode=lax.GatherScatterMode.PROMISE_IN_BOUNDS)

        def body(gb):
            x0 = buf[pl.ds(gb, _L)]
            x1 = buf[pl.ds(gb + _L, _L)]
            x2 = buf[pl.ds(gb + 2 * _L, _L)]
            conf0 = take(x0, i00)
            conf1 = jnp.where(sel1, take(x0, i10), take(x1, i11))
            conf2 = jnp.where(sel2, take(x1, i21), take(x2, i22))
            zero = jnp.float32(0.0)
            obuf[pl.ds(gb, _L)] = jnp.where(conf0 > _THRESHOLD, x0, zero)
            obuf[pl.ds(gb + _L, _L)] = jnp.where(conf1 > _THRESHOLD, x1, zero)
            obuf[pl.ds(gb + 2 * _L, _L)] = jnp.where(conf2 > _THRESHOLD, x2,
                                                     zero)

        plsc.parallel_loop(0, _PER_W, _GROUP, unroll=4)(body)
        pltpu.sync_copy(obuf, out_hbm.at[pl.ds(off, _PER_W)])

    return _filter


_FILTER = _make_kernel()


@jax.jit
def kernel(boxes):
    flat = boxes.reshape(_TOTAL)
    out = _FILTER(flat)
    return out.reshape(_B, _D, _C)


# speedup vs baseline: 1.0380x; 1.0013x over previous
import functools

import numpy as np

import jax
import jax.numpy as jnp
from jax import lax
from jax.experimental import pallas as pl
from jax.experimental.pallas import tpu as pltpu
from jax.experimental.pallas import tpu_sc as plsc

_THRESHOLD = jnp.float32(0.3)

_B, _D, _C = 16, 20000, 6
_TOTAL = _B * _D * _C
_NC, _NS = 2, 16
_NW = _NC * _NS
_PER_W = _TOTAL // _NW
_L = 16
_GROUP = 48


def _make_kernel():
    mesh = plsc.VectorSubcoreMesh(core_axis_name="c", subcore_axis_name="s")

    @functools.partial(
        pl.kernel,
        mesh=mesh,
        out_type=jax.ShapeDtypeStruct((_NW, _PER_W), jnp.float32),
        scratch_types=[
            pltpu.VMEM((_PER_W,), jnp.float32),
            pltpu.VMEM((_PER_W,), jnp.float32),
        ],
        compiler_params=pltpu.CompilerParams(needs_layout_passes=False,
                                             use_tc_tiling_on_sc=False),
    )
    def _filter(x_hbm, out_hbm, buf, obuf):
        wid = lax.axis_index("s") * _NC + lax.axis_index("c")
        pltpu.sync_copy(x_hbm.at[wid], buf)

        lane = lax.iota(jnp.int32, _L)

        def cidx(k, src):
            c = 6 * ((lane + 16 * k) // 6) + 1 - _L * src
            return jnp.clip(c, 0, _L - 1)

        i00 = cidx(0, 0)
        i10, i11 = cidx(1, 0), cidx(1, 1)
        i21, i22 = cidx(2, 1), cidx(2, 2)
        sel1 = lane < 2
        sel2 = lane < 4

        def take(x, i):
            return jnp.take_along_axis(
                x, i, axis=0, mode=lax.GatherScatterMode.PROMISE_IN_BOUNDS)

        def body(gb):
            x0 = buf[pl.ds(gb, _L)]
            x1 = buf[pl.ds(gb + _L, _L)]
            x2 = buf[pl.ds(gb + 2 * _L, _L)]
            conf0 = take(x0, i00)
            conf1 = jnp.where(sel1, take(x0, i10), take(x1, i11))
            conf2 = jnp.where(sel2, take(x1, i21), take(x2, i22))
            zero = jnp.float32(0.0)
            obuf[pl.ds(gb, _L)] = jnp.where(conf0 > _THRESHOLD, x0, zero)
            obuf[pl.ds(gb + _L, _L)] = jnp.where(conf1 > _THRESHOLD, x1, zero)
            obuf[pl.ds(gb + 2 * _L, _L)] = jnp.where(conf2 > _THRESHOLD, x2,
                                                     zero)

        plsc.parallel_loop(0, _PER_W, _GROUP, unroll=4)(body)
        pltpu.sync_copy(obuf, out_hbm.at[wid])

    return _filter


_FILTER = _make_kernel()


@jax.jit
def kernel(boxes):
    out = _FILTER(boxes.reshape(_NW, _PER_W))
    return out.reshape(_B, _D, _C)


# speedup vs baseline: 13.6061x; 13.1081x over previous
import functools

import jax
import jax.numpy as jnp
from jax import lax
from jax.experimental import pallas as pl
from jax.experimental.pallas import tpu as pltpu
from jax.experimental.pallas import tpu_sc as plsc

_THRESHOLD = jnp.float32(0.3)

_B, _D, _C = 16, 20000, 6
_ROWS = _B * _C
_NC, _NS = 2, 16
_NW = _NC * _NS
_L = 16


def _make_kernel():
    mesh = plsc.VectorSubcoreMesh(core_axis_name="c", subcore_axis_name="s")

    @functools.partial(
        pl.kernel,
        mesh=mesh,
        out_type=jax.ShapeDtypeStruct((_ROWS, _D), jnp.float32),
        scratch_types=[
            pltpu.VMEM((_D,), jnp.float32),
            pltpu.VMEM((_D,), jnp.float32),
        ],
        compiler_params=pltpu.CompilerParams(needs_layout_passes=False,
                                             use_tc_tiling_on_sc=True),
    )
    def _filter(x_hbm, out_hbm, cbuf, dbuf):
        wid = lax.axis_index("s") * _NC + lax.axis_index("c")
        b = wid // 2
        h = wid % 2
        pltpu.sync_copy(x_hbm.at[_B + b], cbuf)

        def mask_row(i):
            v = dbuf[pl.ds(i, _L)]
            cf = cbuf[pl.ds(i, _L)]
            dbuf[pl.ds(i, _L)] = jnp.where(cf > _THRESHOLD, v,
                                           jnp.float32(0.0))

        for j in range(3):
            row = 48 * h + 16 * j + b
            pltpu.sync_copy(x_hbm.at[row], dbuf)
            plsc.parallel_loop(0, _D, _L, unroll=8)(mask_row)
            pltpu.sync_copy(dbuf, out_hbm.at[row])

    return _filter


_FILTER = _make_kernel()


@jax.jit
def kernel(boxes):
    rows = boxes.transpose(2, 0, 1).reshape(_ROWS, _D)
    out = _FILTER(rows)
    return out.reshape(_C, _B, _D).transpose(1, 2, 0)


# speedup vs baseline: 15.5075x; 1.1397x over previous
import functools

import jax
import jax.numpy as jnp
from jax import lax
from jax.experimental import pallas as pl
from jax.experimental.pallas import tpu as pltpu
from jax.experimental.pallas import tpu_sc as plsc

_THRESHOLD = jnp.float32(0.3)

_B, _D, _C = 16, 20000, 6
_ROWS = _B * _C
_NC, _NS = 2, 16
_NW = _NC * _NS
_L = 16


def _make_kernel():
    mesh = plsc.VectorSubcoreMesh(core_axis_name="c", subcore_axis_name="s")

    @functools.partial(
        pl.kernel,
        mesh=mesh,
        out_type=jax.ShapeDtypeStruct((_ROWS, _D), jnp.float32),
        scratch_types=[
            pltpu.VMEM((_D,), jnp.float32),
            pltpu.VMEM((_D,), jnp.float32),
            pltpu.VMEM((_D,), jnp.float32),
            pltpu.SemaphoreType.DMA,
            pltpu.SemaphoreType.DMA,
            pltpu.SemaphoreType.DMA,
            pltpu.SemaphoreType.DMA,
        ],
        compiler_params=pltpu.CompilerParams(needs_layout_passes=False,
                                             use_tc_tiling_on_sc=True),
    )
    def _filter(x_hbm, out_hbm, cbuf, dbuf0, dbuf1, csem, isem, osem0, osem1):
        wid = lax.axis_index("s") * _NC + lax.axis_index("c")
        b = wid // 2
        h = wid % 2
        rows = [48 * h + 16 * j + b for j in range(3)]
        dbufs = [dbuf0, dbuf1]
        osems = [osem0, osem1]

        cp_conf = pltpu.make_async_copy(x_hbm.at[_B + b], cbuf, csem)
        cp_conf.start()
        cp_in = pltpu.make_async_copy(x_hbm.at[rows[0]], dbuf0, isem)
        cp_in.start()
        cp_conf.wait()

        def make_mask_row(dbuf):
            def mask_row(i):
                v = dbuf[pl.ds(i, _L)]
                cf = cbuf[pl.ds(i, _L)]
                dbuf[pl.ds(i, _L)] = jnp.where(cf > _THRESHOLD, v,
                                               jnp.float32(0.0))
            return mask_row

        outs = [None, None, None]
        for j in range(3):
            cur = dbufs[j % 2]
            pltpu.make_async_copy(x_hbm.at[rows[j]], cur, isem).wait()
            if j < 2:
                nxt = dbufs[(j + 1) % 2]
                if j + 1 >= 2:
                    outs[j - 1].wait()
                pltpu.make_async_copy(x_hbm.at[rows[j + 1]], nxt, isem).start()
            plsc.parallel_loop(0, _D, _L, unroll=8)(make_mask_row(cur))
            outs[j] = pltpu.make_async_copy(cur, out_hbm.at[rows[j]],
                                            osems[j % 2])
            outs[j].start()
        outs[1].wait()
        outs[2].wait()

    return _filter


_FILTER = _make_kernel()


@jax.jit
def kernel(boxes):
    rows = boxes.transpose(2, 0, 1).reshape(_ROWS, _D)
    out = _FILTER(rows)
    return out.reshape(_C, _B, _D).transpose(1, 2, 0)


# speedup vs baseline: 15.5885x; 1.0052x over previous
import functools

import jax
import jax.numpy as jnp
from jax import lax
from jax.experimental import pallas as pl
from jax.experimental.pallas import tpu as pltpu
from jax.experimental.pallas import tpu_sc as plsc

_THRESHOLD = jnp.float32(0.3)

_B, _D, _C = 16, 20000, 6
_ROWS = _B * _C
_NC, _NS = 2, 16
_NW = _NC * _NS
_L = 16


def _make_kernel():
    mesh = plsc.VectorSubcoreMesh(core_axis_name="c", subcore_axis_name="s")

    @functools.partial(
        pl.kernel,
        mesh=mesh,
        out_type=jax.ShapeDtypeStruct((_ROWS, _D), jnp.float32),
        scratch_types=[
            pltpu.VMEM((_D,), jnp.float32),
            pltpu.VMEM((_D,), jnp.float32),
            pltpu.VMEM((_D,), jnp.float32),
            pltpu.SemaphoreType.DMA,
            pltpu.SemaphoreType.DMA,
            pltpu.SemaphoreType.DMA,
            pltpu.SemaphoreType.DMA,
        ],
        compiler_params=pltpu.CompilerParams(needs_layout_passes=False,
                                             use_tc_tiling_on_sc=True,
                                             skip_device_barrier=True),
    )
    def _filter(x_hbm, out_hbm, cbuf, dbuf0, dbuf1, csem, isem, osem0, osem1):
        wid = lax.axis_index("s") * _NC + lax.axis_index("c")
        b = wid // 2
        h = wid % 2
        rows = [48 * h + 16 * j + b for j in range(3)]
        dbufs = [dbuf0, dbuf1]
        osems = [osem0, osem1]

        cp_conf = pltpu.make_async_copy(x_hbm.at[_B + b], cbuf, csem)
        cp_conf.start()
        cp_in = pltpu.make_async_copy(x_hbm.at[rows[0]], dbuf0, isem)
        cp_in.start()
        cp_conf.wait()

        def make_mask_row(dbuf):
            def mask_row(i):
                v = dbuf[pl.ds(i, _L)]
                cf = cbuf[pl.ds(i, _L)]
                dbuf[pl.ds(i, _L)] = jnp.where(cf > _THRESHOLD, v,
                                               jnp.float32(0.0))
            return mask_row

        outs = [None, None, None]
        for j in range(3):
            cur = dbufs[j % 2]
            pltpu.make_async_copy(x_hbm.at[rows[j]], cur, isem).wait()
            if j < 2:
                nxt = dbufs[(j + 1) % 2]
                if j + 1 >= 2:
                    outs[j - 1].wait()
                pltpu.make_async_copy(x_hbm.at[rows[j + 1]], nxt, isem).start()
            plsc.parallel_loop(0, _D, _L, unroll=8)(make_mask_row(cur))
            outs[j] = pltpu.make_async_copy(cur, out_hbm.at[rows[j]],
                                            osems[j % 2])
            outs[j].start()
        outs[1].wait()
        outs[2].wait()

    return _filter


_FILTER = _make_kernel()


@jax.jit
def kernel(boxes):
    rows = boxes.transpose(2, 0, 1).reshape(_ROWS, _D)
    out = _FILTER(rows)
    return out.reshape(_C, _B, _D).transpose(1, 2, 0)
